# baseline (speedup 1.0000x reference)
import functools

import jax
import jax.numpy as jnp
from jax import lax
from jax.experimental import pallas as pl
from jax.experimental.pallas import tpu as pltpu
from jax.experimental.pallas import tpu_sc as plsc

NC = 2
NS = 16
NW = NC * NS
CHUNK = 40
KPT = 250
NPAD = 10240


def _mesh():
    return plsc.VectorSubcoreMesh(
        core_axis_name="c", subcore_axis_name="s",
        num_cores=NC, num_subcores=NS)


@functools.lru_cache(maxsize=None)
def _deg_kernel(E, G=25):
    K = KPT
    S = K // G
    rows_out = NPAD // NS

    @functools.partial(
        pl.kernel,
        out_type=jax.ShapeDtypeStruct((NC * NPAD,), jnp.float32),
        mesh=_mesh(),
        compiler_params=pltpu.CompilerParams(use_tc_tiling_on_sc=False),
        scratch_types=[
            pltpu.VMEM((K, CHUNK), jnp.int32),
            pltpu.VMEM((CHUNK,), jnp.float32),
            pltpu.VMEM_SHARED((NPAD,), jnp.float32),
            pltpu.SemaphoreType.DMA,
        ],
    )
    def deg(edge_hbm, zeros_hbm, out_hbm, colv, ones_v, acc, ssem):
        cid = lax.axis_index("c")
        sid = lax.axis_index("s")
        wid = cid * NS + sid

        @pl.when(sid == 0)
        def _():
            pltpu.sync_copy(zeros_hbm, acc)

        pltpu.sync_copy(edge_hbm.at[NW + wid], colv)
        for i in range(CHUNK // 16):
            ones_v[pl.ds(i * 16, 16)] = jnp.ones((16,), jnp.float32)
        plsc.subcore_barrier()

        def superstep(s, carry):
            base = s * G
            cps = [pltpu.async_copy(ones_v, acc.at[colv.at[base + b]],
                                    ssem, add=True) for b in range(G)]
            for c in cps:
                c.wait()
            return carry

        lax.fori_loop(0, S, superstep, 0)
        plsc.subcore_barrier()
        src = pl.multiple_of(sid * rows_out, 8)
        dst = pl.multiple_of(cid * NPAD + sid * rows_out, 8)
        pltpu.sync_copy(acc.at[pl.ds(src, rows_out)],
                        out_hbm.at[pl.ds(dst, rows_out)])

    return deg


@functools.lru_cache(maxsize=None)
def _agg_kernel(E, F, G):
    K = KPT
    S = K // G
    RPT = NPAD // NS

    @functools.partial(
        pl.kernel,
        out_type=jax.ShapeDtypeStruct((NC * NPAD, F), jnp.float32),
        mesh=_mesh(),
        compiler_params=pltpu.CompilerParams(use_tc_tiling_on_sc=False),
        scratch_types=[
            pltpu.VMEM((K, CHUNK), jnp.int32),
            pltpu.VMEM((K, CHUNK), jnp.int32),
            pltpu.VMEM((2 * G, CHUNK, F), jnp.float32),
            pltpu.VMEM_SHARED((NPAD, F), jnp.float32),
            pltpu.SemaphoreType.DMA,
            pltpu.SemaphoreType.DMA,
            pltpu.SemaphoreType.DMA,
            pltpu.SemaphoreType.DMA,
        ],
    )
    def agg(y_hbm, edge_hbm, zeros_hbm, out_hbm,
            rowv, colv, bufs, acc, gsem0, gsem1, ssem0, ssem1):
        cid = lax.axis_index("c")
        sid = lax.axis_index("s")
        wid = cid * NS + sid
        gsems = (gsem0, gsem1)
        ssems = (ssem0, ssem1)

        @pl.when(sid == 0)
        def _():
            pltpu.sync_copy(zeros_hbm, acc)

        pltpu.sync_copy(edge_hbm.at[wid], rowv)
        pltpu.sync_copy(edge_hbm.at[NW + wid], colv)
        plsc.subcore_barrier()

        def fire_gathers(base, bank):
            return [pltpu.async_copy(y_hbm.at[rowv.at[base + b]],
                                     bufs.at[bank * G + b], gsems[bank])
                    for b in range(G)]

        for c in fire_gathers(0, 0):
            pass

        def superstep(s, carry):
            def stage(bank):
                base = s * G
                for b in range(G):
                    pltpu.make_async_copy(
                        y_hbm.at[rowv.at[base + b]],
                        bufs.at[bank * G + b], gsems[bank]).wait()
                puts = [pltpu.async_copy(bufs.at[bank * G + b],
                                         acc.at[colv.at[base + b]],
                                         ssems[bank], add=True)
                        for b in range(G)]
                @pl.when(s + 1 < S)
                def _():
                    fire_gathers((s + 1) * G, 1 - bank)
                for c in puts:
                    c.wait()

            @pl.when(s % 2 == 0)
            def _():
                stage(0)

            @pl.when(s % 2 == 1)
            def _():
                stage(1)

            return carry

        lax.fori_loop(0, S, superstep, 0)
        plsc.subcore_barrier()
        src = pl.multiple_of(sid * RPT, 8)
        dst = pl.multiple_of(cid * NPAD + sid * RPT, 8)
        pltpu.sync_copy(acc.at[pl.ds(src, RPT)],
                        out_hbm.at[pl.ds(dst, RPT)])

    return agg


def _col(wide, n):
    r = wide.shape[0]
    rep = jnp.repeat(wide, 128, axis=0)
    lane = lax.broadcasted_iota(jnp.int32, (r * 128, 128), 1)
    srow = lax.broadcasted_iota(jnp.int32, (r * 128, 128), 0)
    col = jnp.sum(jnp.where(lane == srow % 128, rep, 0.0),
                  axis=1, keepdims=True)
    return col[:n]


def _mm1_body(x_ref, w_ref, dp_ref, y_ref, disw_ref):
    n = x_ref.shape[0]
    half = NPAD // 128
    dp = dp_ref[...]
    deg_w = dp[:half] + dp[half:] + 1.0
    dis_w = lax.rsqrt(deg_w)
    disw_ref[...] = dis_w
    dis = _col(dis_w, n)
    y_ref[...] = jnp.dot(x_ref[...], w_ref[...],
                         preferred_element_type=jnp.float32) * dis


def _mid_body(a_ref, y1_ref, disw_ref, b1_ref, w2_ref, y2_ref):
    n = y1_ref.shape[0]
    a = a_ref[...]
    dis = _col(disw_ref[...], n)
    h = jnp.maximum(
        (a[:n] + a[NPAD:NPAD + n] + y1_ref[...]) * dis + b1_ref[...], 0.0)
    y2_ref[...] = jnp.dot(h, w2_ref[...],
                          preferred_element_type=jnp.float32) * dis


def _fin_body(a_ref, y2_ref, disw_ref, b2_ref, out_ref):
    n = y2_ref.shape[0]
    a = a_ref[...]
    dis = _col(disw_ref[...], n)
    res = (a[:n] + a[NPAD:NPAD + n] + y2_ref[...]) * dis + b2_ref[...]
    out_ref[...] = res[:, :out_ref.shape[1]]


@functools.lru_cache(maxsize=None)
def _mm1_call(N, Fin, F):
    return pl.pallas_call(
        _mm1_body,
        out_shape=[jax.ShapeDtypeStruct((N, F), jnp.float32),
                   jax.ShapeDtypeStruct((NPAD // 128, 128), jnp.float32)])


@functools.lru_cache(maxsize=None)
def _mid_call(N, F, F2):
    return pl.pallas_call(
        _mid_body,
        out_shape=jax.ShapeDtypeStruct((N, F2), jnp.float32))


@functools.lru_cache(maxsize=None)
def _fin_call(N, F2, Fout):
    return pl.pallas_call(
        _fin_body,
        out_shape=jax.ShapeDtypeStruct((N, Fout), jnp.float32))


F2PAD = 16


def kernel(x, edge_index, W1, b1, W2, b2):
    N, Fin = x.shape
    E = edge_index.shape[1]
    F = W1.shape[1]
    Fout = W2.shape[1]

    ei = edge_index.astype(jnp.int32)
    npad_e = NW * KPT * CHUNK - E
    if npad_e == 0:
        edge3 = ei.reshape(2 * NW, KPT, CHUNK)
    else:
        pad_row = jnp.zeros((npad_e,), jnp.int32)
        pad_col = jnp.full((npad_e,), N, jnp.int32)
        edge3 = jnp.concatenate(
            [ei[0], pad_row, ei[1], pad_col]).reshape(2 * NW, KPT, CHUNK)

    degp = _deg_kernel(E)(edge3, jnp.zeros((NPAD,), jnp.float32))

    y1, dis = _mm1_call(N, Fin, F)(x, W1,
                                   degp.reshape(NC * NPAD // 128, 128))

    a1 = _agg_kernel(E, F, 10)(y1, edge3, jnp.zeros((NPAD, F), jnp.float32))

    w2p = jnp.zeros((F, F2PAD), jnp.float32).at[:, :Fout].set(W2)
    y2 = _mid_call(N, F, F2PAD)(a1, y1, dis, b1.reshape(1, F), w2p)

    a2 = _agg_kernel(E, F2PAD, 25)(y2, edge3,
                               jnp.zeros((NPAD, F2PAD), jnp.float32))

    b2p = jnp.zeros((1, F2PAD), jnp.float32).at[0, :Fout].set(b2)
    return _fin_call(N, F2PAD, Fout)(a2, y2, dis, b2p)


# speedup vs baseline: 1.0204x; 1.0204x over previous
import functools

import jax
import jax.numpy as jnp
from jax import lax
from jax.experimental import pallas as pl
from jax.experimental.pallas import tpu as pltpu
from jax.experimental.pallas import tpu_sc as plsc

NC = 2
NS = 16
NW = NC * NS
CHUNK = 80
KPT = 125
NPAD = 10240


def _mesh():
    return plsc.VectorSubcoreMesh(
        core_axis_name="c", subcore_axis_name="s",
        num_cores=NC, num_subcores=NS)


@functools.lru_cache(maxsize=None)
def _deg_kernel(E, G=25):
    K = KPT
    S = K // G
    rows_out = NPAD // NS

    @functools.partial(
        pl.kernel,
        out_type=jax.ShapeDtypeStruct((NC * NPAD,), jnp.float32),
        mesh=_mesh(),
        compiler_params=pltpu.CompilerParams(use_tc_tiling_on_sc=False),
        scratch_types=[
            pltpu.VMEM((K, CHUNK), jnp.int32),
            pltpu.VMEM((CHUNK,), jnp.float32),
            pltpu.VMEM_SHARED((NPAD,), jnp.float32),
            pltpu.SemaphoreType.DMA,
        ],
    )
    def deg(edge_hbm, zeros_hbm, out_hbm, colv, ones_v, acc, ssem):
        cid = lax.axis_index("c")
        sid = lax.axis_index("s")
        wid = cid * NS + sid

        @pl.when(sid == 0)
        def _():
            pltpu.sync_copy(zeros_hbm, acc)

        pltpu.sync_copy(edge_hbm.at[NW + wid], colv)
        for i in range(CHUNK // 16):
            ones_v[pl.ds(i * 16, 16)] = jnp.ones((16,), jnp.float32)
        plsc.subcore_barrier()

        def superstep(s, carry):
            base = s * G
            cps = [pltpu.async_copy(ones_v, acc.at[colv.at[base + b]],
                                    ssem, add=True) for b in range(G)]
            for c in cps:
                c.wait()
            return carry

        lax.fori_loop(0, S, superstep, 0)
        plsc.subcore_barrier()
        src = pl.multiple_of(sid * rows_out, 8)
        dst = pl.multiple_of(cid * NPAD + sid * rows_out, 8)
        pltpu.sync_copy(acc.at[pl.ds(src, rows_out)],
                        out_hbm.at[pl.ds(dst, rows_out)])

    return deg


@functools.lru_cache(maxsize=None)
def _agg_kernel(E, F, G):
    K = KPT
    S = K // G
    RPT = NPAD // NS

    @functools.partial(
        pl.kernel,
        out_type=jax.ShapeDtypeStruct((NC * NPAD, F), jnp.float32),
        mesh=_mesh(),
        compiler_params=pltpu.CompilerParams(use_tc_tiling_on_sc=False),
        scratch_types=[
            pltpu.VMEM((K, CHUNK), jnp.int32),
            pltpu.VMEM((K, CHUNK), jnp.int32),
            pltpu.VMEM((2 * G, CHUNK, F), jnp.float32),
            pltpu.VMEM_SHARED((NPAD, F), jnp.float32),
            pltpu.SemaphoreType.DMA,
            pltpu.SemaphoreType.DMA,
            pltpu.SemaphoreType.DMA,
            pltpu.SemaphoreType.DMA,
        ],
    )
    def agg(y_hbm, edge_hbm, zeros_hbm, out_hbm,
            rowv, colv, bufs, acc, gsem0, gsem1, ssem0, ssem1):
        cid = lax.axis_index("c")
        sid = lax.axis_index("s")
        wid = cid * NS + sid
        gsems = (gsem0, gsem1)
        ssems = (ssem0, ssem1)

        @pl.when(sid == 0)
        def _():
            pltpu.sync_copy(zeros_hbm, acc)

        pltpu.sync_copy(edge_hbm.at[wid], rowv)
        pltpu.sync_copy(edge_hbm.at[NW + wid], colv)
        plsc.subcore_barrier()

        def fire_gathers(base, bank):
            return [pltpu.async_copy(y_hbm.at[rowv.at[base + b]],
                                     bufs.at[bank * G + b], gsems[bank])
                    for b in range(G)]

        for c in fire_gathers(0, 0):
            pass

        def superstep(s, carry):
            def stage(bank):
                base = s * G
                for b in range(G):
                    pltpu.make_async_copy(
                        y_hbm.at[rowv.at[base + b]],
                        bufs.at[bank * G + b], gsems[bank]).wait()
                puts = [pltpu.async_copy(bufs.at[bank * G + b],
                                         acc.at[colv.at[base + b]],
                                         ssems[bank], add=True)
                        for b in range(G)]
                @pl.when(s + 1 < S)
                def _():
                    fire_gathers((s + 1) * G, 1 - bank)
                for c in puts:
                    c.wait()

            @pl.when(s % 2 == 0)
            def _():
                stage(0)

            @pl.when(s % 2 == 1)
            def _():
                stage(1)

            return carry

        lax.fori_loop(0, S, superstep, 0)
        plsc.subcore_barrier()
        src = pl.multiple_of(sid * RPT, 8)
        dst = pl.multiple_of(cid * NPAD + sid * RPT, 8)
        pltpu.sync_copy(acc.at[pl.ds(src, RPT)],
                        out_hbm.at[pl.ds(dst, RPT)])

    return agg


def _col(wide, n):
    r = wide.shape[0]
    rep = jnp.repeat(wide, 128, axis=0)
    lane = lax.broadcasted_iota(jnp.int32, (r * 128, 128), 1)
    srow = lax.broadcasted_iota(jnp.int32, (r * 128, 128), 0)
    col = jnp.sum(jnp.where(lane == srow % 128, rep, 0.0),
                  axis=1, keepdims=True)
    return col[:n]


def _mm1_body(x_ref, w_ref, dp_ref, y_ref, disw_ref):
    n = x_ref.shape[0]
    half = NPAD // 128
    dp = dp_ref[...]
    deg_w = dp[:half] + dp[half:] + 1.0
    dis_w = lax.rsqrt(deg_w)
    disw_ref[...] = dis_w
    dis = _col(dis_w, n)
    y_ref[...] = jnp.dot(x_ref[...], w_ref[...],
                         preferred_element_type=jnp.float32) * dis


def _mid_body(a_ref, y1_ref, disw_ref, b1_ref, w2_ref, y2_ref):
    n = y1_ref.shape[0]
    a = a_ref[...]
    dis = _col(disw_ref[...], n)
    h = jnp.maximum(
        (a[:n] + a[NPAD:NPAD + n] + y1_ref[...]) * dis + b1_ref[...], 0.0)
    y2_ref[...] = jnp.dot(h, w2_ref[...],
                          preferred_element_type=jnp.float32) * dis


def _fin_body(a_ref, y2_ref, disw_ref, b2_ref, out_ref):
    n = y2_ref.shape[0]
    a = a_ref[...]
    dis = _col(disw_ref[...], n)
    res = (a[:n] + a[NPAD:NPAD + n] + y2_ref[...]) * dis + b2_ref[...]
    out_ref[...] = res[:, :out_ref.shape[1]]


@functools.lru_cache(maxsize=None)
def _mm1_call(N, Fin, F):
    return pl.pallas_call(
        _mm1_body,
        out_shape=[jax.ShapeDtypeStruct((N, F), jnp.float32),
                   jax.ShapeDtypeStruct((NPAD // 128, 128), jnp.float32)])


@functools.lru_cache(maxsize=None)
def _mid_call(N, F, F2):
    return pl.pallas_call(
        _mid_body,
        out_shape=jax.ShapeDtypeStruct((N, F2), jnp.float32))


@functools.lru_cache(maxsize=None)
def _fin_call(N, F2, Fout):
    return pl.pallas_call(
        _fin_body,
        out_shape=jax.ShapeDtypeStruct((N, Fout), jnp.float32))


F2PAD = 16


def kernel(x, edge_index, W1, b1, W2, b2):
    N, Fin = x.shape
    E = edge_index.shape[1]
    F = W1.shape[1]
    Fout = W2.shape[1]

    ei = edge_index.astype(jnp.int32)
    npad_e = NW * KPT * CHUNK - E
    if npad_e == 0:
        edge3 = ei.reshape(2 * NW, KPT, CHUNK)
    else:
        pad_row = jnp.zeros((npad_e,), jnp.int32)
        pad_col = jnp.full((npad_e,), N, jnp.int32)
        edge3 = jnp.concatenate(
            [ei[0], pad_row, ei[1], pad_col]).reshape(2 * NW, KPT, CHUNK)

    degp = _deg_kernel(E)(edge3, jnp.zeros((NPAD,), jnp.float32))

    y1, dis = _mm1_call(N, Fin, F)(x, W1,
                                   degp.reshape(NC * NPAD // 128, 128))

    a1 = _agg_kernel(E, F, 5)(y1, edge3, jnp.zeros((NPAD, F), jnp.float32))

    w2p = jnp.zeros((F, F2PAD), jnp.float32).at[:, :Fout].set(W2)
    y2 = _mid_call(N, F, F2PAD)(a1, y1, dis, b1.reshape(1, F), w2p)

    a2 = _agg_kernel(E, F2PAD, 25)(y2, edge3,
                               jnp.zeros((NPAD, F2PAD), jnp.float32))

    b2p = jnp.zeros((1, F2PAD), jnp.float32).at[0, :Fout].set(b2)
    return _fin_call(N, F2PAD, Fout)(a2, y2, dis, b2p)
